# baseline (speedup 1.0000x reference)
import functools

import jax
import jax.numpy as jnp
from jax import lax
from jax.experimental import pallas as pl
from jax.experimental.pallas import tpu as pltpu
from jax.experimental.pallas import tpu_sc as plsc


_CHUNK = 80
_CHUNKS = (12800, 38400, 51200, 51200, 51200)
_H1 = 64
_H2 = 32
_NT = 6


def _make_sc_gather(V, D, CL):
    info = plsc.get_sparse_core_info()
    NC, NS = info.num_cores, info.num_subcores
    NW = NC * NS
    assert CL % (NW * _CHUNK) == 0
    per_w = CL // NW
    n_chunks = per_w // _CHUNK

    mesh = plsc.VectorSubcoreMesh(core_axis_name="c", subcore_axis_name="s")

    @functools.partial(
        pl.kernel,
        mesh=mesh,
        out_type=jax.ShapeDtypeStruct((CL, D), jnp.float32),
        scratch_types=[
            pltpu.VMEM((n_chunks, _CHUNK), jnp.int32),
            pltpu.VMEM((_CHUNK, D), jnp.float32),
            pltpu.VMEM((_CHUNK, D), jnp.float32),
            pltpu.SemaphoreType.DMA,
            pltpu.SemaphoreType.DMA,
        ],
    )
    def gather_kernel(table_hbm, ids_hbm, out_hbm, idx_v, buf_a, buf_b, sem_a, sem_b):
        wid = lax.axis_index("s") * NC + lax.axis_index("c")
        pltpu.sync_copy(ids_hbm.at[wid], idx_v)
        base = wid * per_w

        def gstart(j, buf, sem):
            pltpu.async_copy(table_hbm.at[idx_v.at[j]], buf, sem)

        def gwait(j, buf, sem):
            pltpu.make_async_copy(table_hbm.at[idx_v.at[j]], buf, sem).wait()

        def store(j, buf):
            pltpu.sync_copy(buf, out_hbm.at[pl.ds(base + j * _CHUNK, _CHUNK), :])

        gstart(0, buf_a, sem_a)

        def step(i, carry):
            j0 = 2 * i
            j1 = j0 + 1
            gwait(j0, buf_a, sem_a)
            gstart(j1, buf_b, sem_b)
            store(j0, buf_a)
            gwait(j1, buf_b, sem_b)

            @pl.when(j0 + 2 < n_chunks)
            def _():
                gstart(j0 + 2, buf_a, sem_a)

            store(j1, buf_b)
            return carry

        lax.fori_loop(0, n_chunks // 2, step, 0)

        if n_chunks % 2:
            last = n_chunks - 1
            gwait(last, buf_a, sem_a)
            store(last, buf_a)

    return gather_kernel


def _dense_body(g_ref, mk_ref, w1_ref, b1_ref, Wc_ref, pos_ref,
                lng_ref, lnb_ref, *rest):
    out_ref = rest[-1]
    x = g_ref[...]
    R, D = x.shape
    H = _H1 + _H2

    mk = mk_ref[...]
    bw = mk[0:1, :]
    sg = mk[1:2, :]
    rank1 = mk[2:5, :]
    phyf = mk[5:6, :]

    drive = jnp.where(lax.broadcasted_iota(jnp.int32, (H, 1), 0) < _H1, bw, sg)
    hid = jnp.maximum(w1_ref[...] * drive + b1_ref[...], 0.0)

    oh = (phyf == lax.broadcasted_iota(jnp.int32, (_NT, 1), 0).astype(jnp.float32)
          ).astype(jnp.float32)

    ones = jnp.full((1, R), 1.0, jnp.float32)
    F = jnp.concatenate([hid, oh, rank1, ones], axis=0)

    x = x + lax.dot_general(F, Wc_ref[...], (((0,), (0,)), ((), ())),
                            preferred_element_type=jnp.float32)
    x = x + pos_ref[...]

    onesD = jnp.full((D, 1), 1.0 / D, jnp.float32)
    mu = jnp.dot(x, onesD, preferred_element_type=jnp.float32)
    ms = jnp.dot(x * x, onesD, preferred_element_type=jnp.float32)
    var = ms - mu * mu
    s = lax.rsqrt(var + 1e-5) * lng_ref[...][None, :]
    out_ref[...] = (x - mu) * s + lnb_ref[...][None, :]


def _dense_call(g2, mk, w1cat, b1cat, Wcat, pos_tiled, lng, lnb,
                prev, blk0, BL, R):
    CL, D = g2.shape

    def rep2(i):
        return (0, 0)

    def rep1(i):
        return (0,)

    vecD = pl.BlockSpec(lnb.shape, rep1)
    in_specs = [
        pl.BlockSpec((R, D), lambda i: (i, 0)),
        pl.BlockSpec((mk.shape[0], R), lambda i, b=blk0: (0, i + b)),
        pl.BlockSpec(w1cat.shape, rep2), pl.BlockSpec(b1cat.shape, rep2),
        pl.BlockSpec(Wcat.shape, rep2),
        pl.BlockSpec((R, D), rep2),
        vecD, vecD,
    ]
    args = [g2, mk, w1cat, b1cat, Wcat, pos_tiled, lng, lnb]
    io_aliases = {}
    if prev is not None:
        in_specs.append(pl.BlockSpec(memory_space=pltpu.MemorySpace.HBM))
        args.append(prev)
        io_aliases = {len(args) - 1: 0}

    return pl.pallas_call(
        _dense_body,
        grid=(CL // R,),
        in_specs=in_specs,
        out_specs=pl.BlockSpec((R, D), lambda i, b=blk0: (i + b, 0)),
        out_shape=jax.ShapeDtypeStruct((BL, D), jnp.float32),
        input_output_aliases=io_aliases,
        compiler_params=pltpu.CompilerParams(
            dimension_semantics=("arbitrary",)),
    )(*args)


def kernel(token_ids, bitwidths, signed_flags, phy_types, target_mask,
           dependency_mask, fanout_mask, token_table, bw_W1, bw_b1, bw_W2,
           bw_b2, sg_W1, sg_b1, sg_W2, sg_b2, type_table, tgt_W, tgt_b,
           dep_W, dep_b, fan_W, fan_b, pos_table, ln_g, ln_b):
    B, L = token_ids.shape
    V, D = token_table.shape
    BL = B * L
    assert sum(_CHUNKS) == BL

    info = plsc.get_sparse_core_info()
    nw = info.num_cores * info.num_subcores
    ids_flat = token_ids.astype(jnp.int32).reshape(-1)

    def row(a):
        return a.reshape(1, BL).astype(jnp.float32)

    mk = jnp.concatenate([row(bitwidths), row(signed_flags), row(target_mask),
                          row(dependency_mask), row(fanout_mask),
                          row(phy_types)], axis=0)

    w1cat = jnp.concatenate([bw_W1.reshape(-1), sg_W1.reshape(-1)]).reshape(-1, 1)
    b1cat = jnp.concatenate([bw_b1, sg_b1]).reshape(-1, 1)
    bias_total = bw_b2 + sg_b2 + tgt_b + dep_b + fan_b
    Wcat = jnp.concatenate([bw_W2, sg_W2, type_table, tgt_W, dep_W,
                            fan_W, bias_total[None, :]], axis=0)

    gathers = {}
    pos_cache = {}
    gs = []
    start = 0
    for CL in _CHUNKS:
        if CL not in gathers:
            gathers[CL] = _make_sc_gather(V, D, CL)
        ids_c = lax.slice(ids_flat, (start,), (start + CL,)).reshape(
            nw, CL // (nw * _CHUNK), _CHUNK)
        gs.append(gathers[CL](token_table, ids_c))
        start += CL

    buf = None
    start = 0
    for c, CL in enumerate(_CHUNKS):
        R = 3200 if CL <= 12800 else 6400
        assert R % L == 0 and CL % R == 0 and start % R == 0 and start % L == 0
        if R not in pos_cache:
            pos_cache[R] = jnp.tile(pos_table[:L], (R // L, 1))
        buf = _dense_call(gs[c], mk, w1cat, b1cat, Wcat, pos_cache[R],
                          ln_g, ln_b, prev=buf, blk0=start // R, BL=BL, R=R)
        start += CL
    return buf.reshape(B, L, D)


# speedup vs baseline: 1.1080x; 1.1080x over previous
import functools

import jax
import jax.numpy as jnp
from jax import lax
from jax.experimental import pallas as pl
from jax.experimental.pallas import tpu as pltpu
from jax.experimental.pallas import tpu_sc as plsc


_CHUNK = 128
_CHUNKS = (102400, 102400)
_H1 = 64
_H2 = 32
_NT = 6


def _make_sc_gather(V, D, CL):
    info = plsc.get_sparse_core_info()
    NC, NS = info.num_cores, info.num_subcores
    NW = NC * NS
    assert CL % (NW * _CHUNK) == 0
    per_w = CL // NW
    n_chunks = per_w // _CHUNK

    mesh = plsc.VectorSubcoreMesh(core_axis_name="c", subcore_axis_name="s")

    @functools.partial(
        pl.kernel,
        mesh=mesh,
        out_type=jax.ShapeDtypeStruct((CL, D), jnp.float32),
        scratch_types=[
            pltpu.VMEM((n_chunks, _CHUNK), jnp.int32),
            pltpu.VMEM((_CHUNK, D), jnp.float32),
            pltpu.VMEM((_CHUNK, D), jnp.float32),
            pltpu.SemaphoreType.DMA,
            pltpu.SemaphoreType.DMA,
        ],
    )
    def gather_kernel(table_hbm, ids_hbm, out_hbm, idx_v, buf_a, buf_b, sem_a, sem_b):
        wid = lax.axis_index("s") * NC + lax.axis_index("c")
        pltpu.sync_copy(ids_hbm.at[wid], idx_v)
        base = wid * per_w

        def gstart(j, buf, sem):
            pltpu.async_copy(table_hbm.at[idx_v.at[j]], buf, sem)

        def gwait(j, buf, sem):
            pltpu.make_async_copy(table_hbm.at[idx_v.at[j]], buf, sem).wait()

        def store(j, buf):
            pltpu.sync_copy(buf, out_hbm.at[pl.ds(base + j * _CHUNK, _CHUNK), :])

        gstart(0, buf_a, sem_a)

        def step(i, carry):
            j0 = 2 * i
            j1 = j0 + 1
            gwait(j0, buf_a, sem_a)
            gstart(j1, buf_b, sem_b)
            store(j0, buf_a)
            gwait(j1, buf_b, sem_b)

            @pl.when(j0 + 2 < n_chunks)
            def _():
                gstart(j0 + 2, buf_a, sem_a)

            store(j1, buf_b)
            return carry

        lax.fori_loop(0, n_chunks // 2, step, 0)

        if n_chunks % 2:
            last = n_chunks - 1
            gwait(last, buf_a, sem_a)
            store(last, buf_a)

    return gather_kernel


def _dense_body(g_ref, mk_ref, w1_ref, b1_ref, Wc_ref, pos_ref,
                lng_ref, lnb_ref, *rest):
    out_ref = rest[-1]
    x = g_ref[...]
    R, D = x.shape
    H = _H1 + _H2

    mk = mk_ref[...]
    bw = mk[0:1, :]
    sg = mk[1:2, :]
    rank1 = mk[2:5, :]
    phyf = mk[5:6, :]

    drive = jnp.where(lax.broadcasted_iota(jnp.int32, (H, 1), 0) < _H1, bw, sg)
    hid = jnp.maximum(w1_ref[...] * drive + b1_ref[...], 0.0)

    oh = (phyf == lax.broadcasted_iota(jnp.int32, (_NT, 1), 0).astype(jnp.float32)
          ).astype(jnp.float32)

    ones = jnp.full((1, R), 1.0, jnp.float32)
    F = jnp.concatenate([hid, oh, rank1, ones], axis=0)

    x = x + lax.dot_general(F, Wc_ref[...], (((0,), (0,)), ((), ())),
                            preferred_element_type=jnp.float32)
    x = x + pos_ref[...]

    onesD = jnp.full((D, 1), 1.0 / D, jnp.float32)
    mu = jnp.dot(x, onesD, preferred_element_type=jnp.float32)
    ms = jnp.dot(x * x, onesD, preferred_element_type=jnp.float32)
    var = ms - mu * mu
    s = lax.rsqrt(var + 1e-5) * lng_ref[...][None, :]
    out_ref[...] = (x - mu) * s + lnb_ref[...][None, :]


def _dense_call(g2, mk, w1cat, b1cat, Wcat, pos_tiled, lng, lnb,
                prev, blk0, BL, R):
    CL, D = g2.shape

    def rep2(i):
        return (0, 0)

    def rep1(i):
        return (0,)

    vecD = pl.BlockSpec(lnb.shape, rep1)
    in_specs = [
        pl.BlockSpec((R, D), lambda i: (i, 0)),
        pl.BlockSpec((mk.shape[0], R), lambda i, b=blk0: (0, i + b)),
        pl.BlockSpec(w1cat.shape, rep2), pl.BlockSpec(b1cat.shape, rep2),
        pl.BlockSpec(Wcat.shape, rep2),
        pl.BlockSpec((R, D), rep2),
        vecD, vecD,
    ]
    args = [g2, mk, w1cat, b1cat, Wcat, pos_tiled, lng, lnb]
    io_aliases = {}
    if prev is not None:
        in_specs.append(pl.BlockSpec(memory_space=pltpu.MemorySpace.HBM))
        args.append(prev)
        io_aliases = {len(args) - 1: 0}

    return pl.pallas_call(
        _dense_body,
        grid=(CL // R,),
        in_specs=in_specs,
        out_specs=pl.BlockSpec((R, D), lambda i, b=blk0: (i + b, 0)),
        out_shape=jax.ShapeDtypeStruct((BL, D), jnp.float32),
        input_output_aliases=io_aliases,
        compiler_params=pltpu.CompilerParams(
            dimension_semantics=("arbitrary",)),
    )(*args)


def kernel(token_ids, bitwidths, signed_flags, phy_types, target_mask,
           dependency_mask, fanout_mask, token_table, bw_W1, bw_b1, bw_W2,
           bw_b2, sg_W1, sg_b1, sg_W2, sg_b2, type_table, tgt_W, tgt_b,
           dep_W, dep_b, fan_W, fan_b, pos_table, ln_g, ln_b):
    B, L = token_ids.shape
    V, D = token_table.shape
    BL = B * L
    assert sum(_CHUNKS) == BL

    info = plsc.get_sparse_core_info()
    nw = info.num_cores * info.num_subcores
    ids_flat = token_ids.astype(jnp.int32).reshape(-1)

    def row(a):
        return a.reshape(1, BL).astype(jnp.float32)

    mk = jnp.concatenate([row(bitwidths), row(signed_flags), row(target_mask),
                          row(dependency_mask), row(fanout_mask),
                          row(phy_types)], axis=0)

    w1cat = jnp.concatenate([bw_W1.reshape(-1), sg_W1.reshape(-1)]).reshape(-1, 1)
    b1cat = jnp.concatenate([bw_b1, sg_b1]).reshape(-1, 1)
    bias_total = bw_b2 + sg_b2 + tgt_b + dep_b + fan_b
    Wcat = jnp.concatenate([bw_W2, sg_W2, type_table, tgt_W, dep_W,
                            fan_W, bias_total[None, :]], axis=0)

    gathers = {}
    pos_cache = {}
    gs = []
    start = 0
    for CL in _CHUNKS:
        if CL not in gathers:
            gathers[CL] = _make_sc_gather(V, D, CL)
        ids_c = lax.slice(ids_flat, (start,), (start + CL,)).reshape(
            nw, CL // (nw * _CHUNK), _CHUNK)
        gs.append(gathers[CL](token_table, ids_c))
        start += CL

    buf = None
    start = 0
    for c, CL in enumerate(_CHUNKS):
        R = 12800 if CL % 12800 == 0 else 3200
        assert R % L == 0 and CL % R == 0 and start % R == 0 and start % L == 0
        if R not in pos_cache:
            pos_cache[R] = jnp.tile(pos_table[:L], (R // L, 1))
        buf = _dense_call(gs[c], mk, w1cat, b1cat, Wcat, pos_cache[R],
                          ln_g, ln_b, prev=buf, blk0=start // R, BL=BL, R=R)
        start += CL
    return buf.reshape(B, L, D)
